# baseline (speedup 1.0000x reference)
import jax
import jax.numpy as jnp
from jax import lax
from jax.experimental import pallas as pl
from jax.experimental.pallas import tpu as pltpu
from jax.experimental.pallas import tpu_sc as plsc

_B, _C, _H, _W = 16, 768, 8, 256
_NCIR = 4
_L = 16
_NW = 32
_CPW = _C // _NW
_NSLOT = 3
_WL = _W // _L


def _sc_body(x_hbm, circ_hbm, out_hbm, xb0, xb1, xb2, circ_v,
             sem_in, sem_o0, sem_o1, sem_o2):
    wid = lax.axis_index("s") * 2 + lax.axis_index("c")
    c0 = wid * _CPW
    bufs = [xb0, xb1, xb2]
    osems = [sem_o0, sem_o1, sem_o2]
    for k in range(_NCIR):
        pltpu.sync_copy(circ_hbm.at[pl.ds((k * _C + c0) * _L, _CPW * _L)],
                        circ_v.at[pl.ds(k * _CPW * _L, _CPW * _L)])

    def fire_in(ci, slot):
        pltpu.async_copy(x_hbm.at[:, c0 + ci], bufs[slot], sem_in)

    def wait_in(slot):
        pltpu.make_async_copy(x_hbm.at[:, c0], bufs[slot], sem_in).wait()

    def fire_out(ci, slot):
        pltpu.async_copy(bufs[slot], out_hbm.at[:, c0 + ci], osems[slot])

    def wait_out(slot):
        pltpu.make_async_copy(bufs[slot], out_hbm.at[:, c0],
                              osems[slot]).wait()

    def sel(d, s):
        return jnp.where(d == 0, s[0],
                         jnp.where(d == 1, s[1],
                                   jnp.where(d == 2, s[2], s[3])))

    def comp(ci, slot):
        buf = bufs[slot]
        s = [circ_v[pl.ds((k * _CPW + ci) * _L, _L)] for k in range(_NCIR)]
        iot = lax.iota(jnp.int32, _L)
        for h in range(_H):
            mh = min(h, _H - 1 - h)
            a_mid = s[mh]
            if mh == 0:
                lo, nmid = 0, _WL // 2
            else:
                a0 = sel(jnp.minimum(mh, iot), s)
                a15 = sel(jnp.minimum(mh, (_L - 1) - iot), s)
                for b in range(_B):
                    plsc.addupdate(buf.at[b, h, pl.ds(0, _L)], a0)
                    plsc.addupdate(buf.at[b, h, pl.ds(_W - _L, _L)], a15)
                lo, nmid = 1, (_WL - 2) // 2

            def mid(t, carry, lo=lo, h=h, a_mid=a_mid, buf=buf):
                for u in range(2):
                    base = (lo + t * 2 + u) * _L
                    for b in range(_B):
                        plsc.addupdate(buf.at[b, h, pl.ds(base, _L)], a_mid)
                return carry
            lax.fori_loop(0, nmid, mid, 0)

    def step(ci, slot, first=False, fire=True):
        wait_in(slot)
        comp(ci, slot)
        fire_out(ci, slot)
        if fire:
            if not first:
                wait_out((slot + 2) % _NSLOT)
            fire_in(ci + 2, (slot + 2) % _NSLOT)

    fire_in(0, 0)
    fire_in(1, 1)
    step(0, 0, first=True)
    step(1, 1)
    step(2, 2)

    def group(g, carry):
        for sl in range(_NSLOT):
            step(g * _NSLOT + sl, sl)
        return carry
    lax.fori_loop(1, _CPW // _NSLOT - 1, group, 0)

    step(_CPW - 3, 0)
    step(_CPW - 2, 1, fire=False)
    step(_CPW - 1, 2, fire=False)
    wait_out(0)
    wait_out(1)
    wait_out(2)


def kernel(x, circles):
    circ_rep = jnp.broadcast_to(
        circles.astype(jnp.float32).reshape(_NCIR * _C, 1), (_NCIR * _C, _L)
    ).reshape(_NCIR * _C * _L)
    mesh = plsc.VectorSubcoreMesh(core_axis_name="c", subcore_axis_name="s")
    run = pl.kernel(
        _sc_body,
        mesh=mesh,
        out_type=jax.ShapeDtypeStruct((_B, _C, _H, _W), jnp.float32),
        scratch_types=[
            pltpu.VMEM((_B, _H, _W), jnp.float32),
            pltpu.VMEM((_B, _H, _W), jnp.float32),
            pltpu.VMEM((_B, _H, _W), jnp.float32),
            pltpu.VMEM((_NCIR * _CPW * _L,), jnp.float32),
            pltpu.SemaphoreType.DMA,
            pltpu.SemaphoreType.DMA,
            pltpu.SemaphoreType.DMA,
            pltpu.SemaphoreType.DMA,
        ],
    )
    return run(x.astype(jnp.float32), circ_rep)


# speedup vs baseline: 1.0733x; 1.0733x over previous
import jax
import jax.numpy as jnp
from jax import lax
from jax.experimental import pallas as pl
from jax.experimental.pallas import tpu as pltpu
from jax.experimental.pallas import tpu_sc as plsc

_B, _C, _H, _W = 16, 768, 8, 256
_HW = _H * _W
_NCIR = 4
_L = 16
_NW = 32
_CPW = _C // _NW
_NSLOT = 3
_WL = _W // _L


def _sc_body(x_hbm, circ_hbm, out_hbm, xb0, xb1, xb2, circ_v,
             sem_in, sem_o0, sem_o1, sem_o2):
    wid = lax.axis_index("s") * 2 + lax.axis_index("c")
    c0 = wid * _CPW
    bufs = [xb0, xb1, xb2]
    osems = [sem_o0, sem_o1, sem_o2]
    for k in range(_NCIR):
        pltpu.sync_copy(circ_hbm.at[pl.ds((k * _C + c0) * _L, _CPW * _L)],
                        circ_v.at[pl.ds(k * _CPW * _L, _CPW * _L)])

    def fire_in(ci):
        return pltpu.async_copy(x_hbm.at[:, c0 + ci],
                                bufs[ci % _NSLOT], sem_in)

    def fire_out(ci):
        return pltpu.async_copy(bufs[ci % _NSLOT],
                                out_hbm.at[:, c0 + ci], osems[ci % _NSLOT])

    iot = lax.iota(jnp.int32, _L)

    def comp(ci):
        buf = bufs[ci % _NSLOT]
        s = [circ_v[pl.ds((k * _CPW + ci) * _L, _L)] for k in range(_NCIR)]

        @plsc.parallel_loop(0, _HW // _L, 1, unroll=2)
        def add(j):
            h = lax.shift_right_logical(j, 4)
            jw = lax.bitwise_and(j, _WL - 1)
            w = jw * _L + iot
            d = jnp.minimum(jnp.minimum(h, (_H - 1) - h),
                            jnp.minimum(w, (_W - 1) - w))
            a = jnp.where(d == 0, s[0],
                          jnp.where(d == 1, s[1],
                                    jnp.where(d == 2, s[2], s[3])))
            for b in range(_B):
                plsc.addupdate(buf.at[b, h, pl.ds(jw * _L, _L)], a)

    in_cp = [None] * _CPW
    out_cp = [None] * _CPW
    in_cp[0] = fire_in(0)
    in_cp[1] = fire_in(1)
    for ci in range(_CPW):
        in_cp[ci].wait()
        comp(ci)
        out_cp[ci] = fire_out(ci)
        if ci + 2 < _CPW:
            if ci >= 1:
                out_cp[ci - 1].wait()
            in_cp[ci + 2] = fire_in(ci + 2)
    out_cp[_CPW - 3].wait()
    out_cp[_CPW - 2].wait()
    out_cp[_CPW - 1].wait()


def kernel(x, circles):
    circ_rep = jnp.broadcast_to(
        circles.astype(jnp.float32).reshape(_NCIR * _C, 1), (_NCIR * _C, _L)
    ).reshape(_NCIR * _C * _L)
    mesh = plsc.VectorSubcoreMesh(core_axis_name="c", subcore_axis_name="s")
    run = pl.kernel(
        _sc_body,
        mesh=mesh,
        out_type=jax.ShapeDtypeStruct((_B, _C, _H, _W), jnp.float32),
        scratch_types=[
            pltpu.VMEM((_B, _H, _W), jnp.float32),
            pltpu.VMEM((_B, _H, _W), jnp.float32),
            pltpu.VMEM((_B, _H, _W), jnp.float32),
            pltpu.VMEM((_NCIR * _CPW * _L,), jnp.float32),
            pltpu.SemaphoreType.DMA,
            pltpu.SemaphoreType.DMA,
            pltpu.SemaphoreType.DMA,
            pltpu.SemaphoreType.DMA,
        ],
    )
    return run(x.astype(jnp.float32), circ_rep)


# speedup vs baseline: 1.0836x; 1.0096x over previous
import jax
import jax.numpy as jnp
from jax import lax
from jax.experimental import pallas as pl
from jax.experimental.pallas import tpu as pltpu
from jax.experimental.pallas import tpu_sc as plsc

_B, _C, _H, _W = 16, 768, 8, 256
_NCIR = 4
_L = 16
_NW = 32
_CPW = _C // _NW
_NSLOT = 3
_WL = _W // _L
_BH = _B // 2
_NU = _CPW


def _sc_body(x_hbm, circ_hbm, out_hbm, xb0, xb1, xb2, circ_v,
             sem_in, sem_o0, sem_o1, sem_o2):
    wid = lax.axis_index("s") * 2 + lax.axis_index("c")
    c0 = wid * _CPW
    bufs = [xb0, xb1, xb2]
    osems = [sem_o0, sem_o1, sem_o2]
    for k in range(_NCIR):
        pltpu.sync_copy(circ_hbm.at[pl.ds((k * _C + c0) * _L, _CPW * _L)],
                        circ_v.at[pl.ds(k * _CPW * _L, _CPW * _L)])

    def unit(ui):
        pi, b0 = ui >> 1, (ui & 1) * _BH
        return pi, b0

    def fire_in(ui):
        pi, b0 = unit(ui)
        return pltpu.async_copy(
            x_hbm.at[pl.ds(b0, _BH), pl.ds(c0 + 2 * pi, 2)],
            bufs[ui % _NSLOT], sem_in)

    def fire_out(ui):
        pi, b0 = unit(ui)
        return pltpu.async_copy(
            bufs[ui % _NSLOT],
            out_hbm.at[pl.ds(b0, _BH), pl.ds(c0 + 2 * pi, 2)],
            osems[ui % _NSLOT])

    iot = lax.iota(jnp.int32, _L)

    def comp(ui):
        pi, _ = unit(ui)
        buf = bufs[ui % _NSLOT]
        s = [[circ_v[pl.ds((k * _CPW + 2 * pi + u) * _L, _L)]
              for k in range(_NCIR)] for u in range(2)]

        def add(j, carry):
            h = lax.shift_right_logical(j, 4)
            jw = lax.bitwise_and(j, _WL - 1)
            w = jw * _L + iot
            d = jnp.minimum(jnp.minimum(h, (_H - 1) - h),
                            jnp.minimum(w, (_W - 1) - w))
            for u in range(2):
                su = s[u]
                a = jnp.where(d == 0, su[0],
                              jnp.where(d == 1, su[1],
                                        jnp.where(d == 2, su[2], su[3])))
                for b in range(_BH):
                    plsc.addupdate(buf.at[b, u, h, pl.ds(jw * _L, _L)], a)
            return carry
        lax.fori_loop(0, _H * _WL, add, 0)

    in_cp = [None] * _NU
    out_cp = [None] * _NU
    in_cp[0] = fire_in(0)
    in_cp[1] = fire_in(1)
    for ui in range(_NU):
        in_cp[ui].wait()
        comp(ui)
        out_cp[ui] = fire_out(ui)
        if ui + 2 < _NU:
            if ui >= 1:
                out_cp[ui - 1].wait()
            in_cp[ui + 2] = fire_in(ui + 2)
    out_cp[_NU - 3].wait()
    out_cp[_NU - 2].wait()
    out_cp[_NU - 1].wait()


def kernel(x, circles):
    circ_rep = jnp.broadcast_to(
        circles.astype(jnp.float32).reshape(_NCIR * _C, 1), (_NCIR * _C, _L)
    ).reshape(_NCIR * _C * _L)
    mesh = plsc.VectorSubcoreMesh(core_axis_name="c", subcore_axis_name="s")
    run = pl.kernel(
        _sc_body,
        mesh=mesh,
        out_type=jax.ShapeDtypeStruct((_B, _C, _H, _W), jnp.float32),
        scratch_types=[
            pltpu.VMEM((_BH, 2, _H, _W), jnp.float32),
            pltpu.VMEM((_BH, 2, _H, _W), jnp.float32),
            pltpu.VMEM((_BH, 2, _H, _W), jnp.float32),
            pltpu.VMEM((_NCIR * _CPW * _L,), jnp.float32),
            pltpu.SemaphoreType.DMA,
            pltpu.SemaphoreType.DMA,
            pltpu.SemaphoreType.DMA,
            pltpu.SemaphoreType.DMA,
        ],
    )
    return run(x.astype(jnp.float32), circ_rep)


# speedup vs baseline: 1.1075x; 1.0221x over previous
import jax
import jax.numpy as jnp
from jax import lax
from jax.experimental import pallas as pl
from jax.experimental.pallas import tpu as pltpu
from jax.experimental.pallas import tpu_sc as plsc

_B, _C, _H, _W = 16, 768, 8, 256
_NCIR = 4
_L = 16
_NW = 32
_CPW = _C // _NW
_NSLOT = 3
_WL = _W // _L
_BH = _B // 2
_NU = _CPW


def _sc_body(x_hbm, circ_hbm, out_hbm, xb0, xb1, xb2, circ_v,
             sem_in, sem_o0, sem_o1, sem_o2):
    wid = lax.axis_index("s") * 2 + lax.axis_index("c")
    c0 = wid * _CPW
    bufs = [xb0, xb1, xb2]
    osems = [sem_o0, sem_o1, sem_o2]
    for k in range(_NCIR):
        pltpu.sync_copy(circ_hbm.at[pl.ds((k * _C + c0) * _L, _CPW * _L)],
                        circ_v.at[pl.ds(k * _CPW * _L, _CPW * _L)])

    def unit(ui):
        pi = lax.shift_right_logical(ui, 1)
        b0 = lax.bitwise_and(ui, 1) * _BH
        return pi, b0

    def fire_in(ui, slot):
        pi, b0 = unit(ui)
        pltpu.async_copy(
            x_hbm.at[pl.ds(b0, _BH), pl.ds(c0 + 2 * pi, 2)],
            bufs[slot], sem_in)

    def wait_in(slot):
        pltpu.make_async_copy(
            x_hbm.at[pl.ds(0, _BH), pl.ds(c0, 2)],
            bufs[slot], sem_in).wait()

    def fire_out(ui, slot):
        pi, b0 = unit(ui)
        pltpu.async_copy(
            bufs[slot],
            out_hbm.at[pl.ds(b0, _BH), pl.ds(c0 + 2 * pi, 2)],
            osems[slot])

    def wait_out(slot):
        pltpu.make_async_copy(
            bufs[slot],
            out_hbm.at[pl.ds(0, _BH), pl.ds(c0, 2)],
            osems[slot]).wait()

    iot = lax.iota(jnp.int32, _L)

    def comp(ui, slot):
        pi, _ = unit(ui)
        buf = bufs[slot]
        s = [[circ_v[pl.ds((k * _CPW + 2 * pi + u) * _L, _L)]
              for k in range(_NCIR)] for u in range(2)]

        def add(j, carry):
            h = lax.shift_right_logical(j, 4)
            jw = lax.bitwise_and(j, _WL - 1)
            w = jw * _L + iot
            d = jnp.minimum(jnp.minimum(h, (_H - 1) - h),
                            jnp.minimum(w, (_W - 1) - w))
            for u in range(2):
                su = s[u]
                a = jnp.where(d == 0, su[0],
                              jnp.where(d == 1, su[1],
                                        jnp.where(d == 2, su[2], su[3])))
                for b in range(_BH):
                    plsc.addupdate(buf.at[b, u, h, pl.ds(jw * _L, _L)], a)
            return carry
        lax.fori_loop(0, _H * _WL, add, 0)

    def step(ui, slot, first=False, fire=True):
        wait_in(slot)
        comp(ui, slot)
        fire_out(ui, slot)
        if fire:
            if not first:
                wait_out((slot + 2) % _NSLOT)
            fire_in(ui + 2, (slot + 2) % _NSLOT)

    fire_in(0, 0)
    fire_in(1, 1)
    step(0, 0, first=True)
    step(1, 1)
    step(2, 2)

    def group(g, carry):
        for sl in range(_NSLOT):
            step(g * _NSLOT + sl, sl)
        return carry
    lax.fori_loop(1, _NU // _NSLOT - 1, group, 0)

    step(_NU - 3, 0)
    step(_NU - 2, 1, fire=False)
    step(_NU - 1, 2, fire=False)
    wait_out(0)
    wait_out(1)
    wait_out(2)


def kernel(x, circles):
    circ_rep = jnp.broadcast_to(
        circles.astype(jnp.float32).reshape(_NCIR * _C, 1), (_NCIR * _C, _L)
    ).reshape(_NCIR * _C * _L)
    mesh = plsc.VectorSubcoreMesh(core_axis_name="c", subcore_axis_name="s")
    run = pl.kernel(
        _sc_body,
        mesh=mesh,
        out_type=jax.ShapeDtypeStruct((_B, _C, _H, _W), jnp.float32),
        scratch_types=[
            pltpu.VMEM((_BH, 2, _H, _W), jnp.float32),
            pltpu.VMEM((_BH, 2, _H, _W), jnp.float32),
            pltpu.VMEM((_BH, 2, _H, _W), jnp.float32),
            pltpu.VMEM((_NCIR * _CPW * _L,), jnp.float32),
            pltpu.SemaphoreType.DMA,
            pltpu.SemaphoreType.DMA,
            pltpu.SemaphoreType.DMA,
            pltpu.SemaphoreType.DMA,
        ],
    )
    return run(x.astype(jnp.float32), circ_rep)


# speedup vs baseline: 1.1184x; 1.0099x over previous
import jax
import jax.numpy as jnp
from jax import lax
from jax.experimental import pallas as pl
from jax.experimental.pallas import tpu as pltpu
from jax.experimental.pallas import tpu_sc as plsc

_B, _C, _H, _W = 16, 768, 8, 256
_NCIR = 4
_L = 16
_NW = 32
_CPW = _C // _NW
_NSLOT = 3
_WL = _W // _L
_BH = _B // 2
_NU = _CPW


def _sc_body(x_hbm, circ_hbm, out_hbm, xb0, xb1, xb2, circ_v,
             sem_in, sem_o0, sem_o1, sem_o2):
    wid = lax.axis_index("s") * 2 + lax.axis_index("c")
    c0 = wid * _CPW
    bufs = [xb0, xb1, xb2]
    osems = [sem_o0, sem_o1, sem_o2]
    for k in range(_NCIR):
        pltpu.sync_copy(circ_hbm.at[pl.ds((k * _C + c0) * _L, _CPW * _L)],
                        circ_v.at[pl.ds(k * _CPW * _L, _CPW * _L)])

    def unit(ui):
        pi = lax.shift_right_logical(ui, 1)
        b0 = lax.bitwise_and(ui, 1) * _BH
        return pi, b0

    def fire_in(ui, slot):
        pi, b0 = unit(ui)
        pltpu.async_copy(
            x_hbm.at[pl.ds(b0, _BH), pl.ds(c0 + 2 * pi, 2)],
            bufs[slot], sem_in)

    def wait_in(slot):
        pltpu.make_async_copy(
            x_hbm.at[pl.ds(0, _BH), pl.ds(c0, 2)],
            bufs[slot], sem_in).wait()

    def fire_out(ui, slot):
        pi, b0 = unit(ui)
        pltpu.async_copy(
            bufs[slot],
            out_hbm.at[pl.ds(b0, _BH), pl.ds(c0 + 2 * pi, 2)],
            osems[slot])

    def wait_out(slot):
        pltpu.make_async_copy(
            bufs[slot],
            out_hbm.at[pl.ds(0, _BH), pl.ds(c0, 2)],
            osems[slot]).wait()

    iot = lax.iota(jnp.int32, _L)

    def comp(ui, slot):
        pi, _ = unit(ui)
        buf = bufs[slot]
        s = [[circ_v[pl.ds((k * _CPW + 2 * pi + u) * _L, _L)]
              for k in range(_NCIR)] for u in range(2)]

        def add(j, carry):
            h = lax.shift_right_logical(j, 4)
            jw = lax.bitwise_and(j, _WL - 1)
            w = jw * _L + iot
            d = jnp.minimum(jnp.minimum(h, (_H - 1) - h),
                            jnp.minimum(w, (_W - 1) - w))
            for u in range(2):
                su = s[u]
                a = jnp.where(d == 0, su[0],
                              jnp.where(d == 1, su[1],
                                        jnp.where(d == 2, su[2], su[3])))
                for b in range(_BH):
                    plsc.addupdate(buf.at[b, u, h, pl.ds(jw * _L, _L)], a)
            return carry
        lax.fori_loop(0, _H * _WL, add, 0)

    def step(ui, slot):
        wait_in(slot)
        comp(ui, slot)
        fire_out(ui, slot)

        @pl.when(jnp.logical_and(ui >= 1, ui + 2 < _NU))
        def _():
            wait_out((slot + 2) % _NSLOT)

        @pl.when(ui + 2 < _NU)
        def _():
            fire_in(ui + 2, (slot + 2) % _NSLOT)

    fire_in(0, 0)
    fire_in(1, 1)

    def group(g, carry):
        for sl in range(_NSLOT):
            step(g * _NSLOT + sl, sl)
        return carry
    lax.fori_loop(0, _NU // _NSLOT, group, 0)

    wait_out(0)
    wait_out(1)
    wait_out(2)


def kernel(x, circles):
    circ_rep = jnp.broadcast_to(
        circles.astype(jnp.float32).reshape(_NCIR * _C, 1), (_NCIR * _C, _L)
    ).reshape(_NCIR * _C * _L)
    mesh = plsc.VectorSubcoreMesh(core_axis_name="c", subcore_axis_name="s")
    run = pl.kernel(
        _sc_body,
        mesh=mesh,
        out_type=jax.ShapeDtypeStruct((_B, _C, _H, _W), jnp.float32),
        scratch_types=[
            pltpu.VMEM((_BH, 2, _H, _W), jnp.float32),
            pltpu.VMEM((_BH, 2, _H, _W), jnp.float32),
            pltpu.VMEM((_BH, 2, _H, _W), jnp.float32),
            pltpu.VMEM((_NCIR * _CPW * _L,), jnp.float32),
            pltpu.SemaphoreType.DMA,
            pltpu.SemaphoreType.DMA,
            pltpu.SemaphoreType.DMA,
            pltpu.SemaphoreType.DMA,
        ],
    )
    return run(x.astype(jnp.float32), circ_rep)
